# baseline (speedup 1.0000x reference)
import jax
import jax.numpy as jnp
from jax import lax
from jax.experimental import pallas as pl
from jax.experimental.pallas import tpu as pltpu
from jax.experimental.pallas import tpu_sc as plsc

B, S, D = 4, 8192, 768
N = B * S
NC, NS = 2, 16
NW = NC * NS
TPW = N // NW
R = 32
NCHUNK = TPW // R
LANES = 16
JCOLS = D // LANES


def _body(ids_hbm, tt_hbm, tok_tab, pos_tab, typ_tab, out_hbm,
          idx_v, ttx_v, tok_v, pos_v, typ_v, sem_tok, sem_typ):
    wid = lax.axis_index("s") * NC + lax.axis_index("c")
    base0 = wid * TPW
    pos0 = base0 % S

    def chunk(c, carry):
        base = base0 + c * R
        pbase = pos0 + c * R
        pltpu.sync_copy(ids_hbm.at[pl.ds(base, R)], idx_v)
        pltpu.sync_copy(tt_hbm.at[pl.ds(base, R)], ttx_v)
        ctok = pltpu.async_copy(tok_tab.at[idx_v], tok_v, sem_tok)
        ctyp = pltpu.async_copy(typ_tab.at[ttx_v], typ_v, sem_typ)
        pltpu.sync_copy(pos_tab.at[pl.ds(pbase, R)], pos_v)
        ctok.wait()
        ctyp.wait()

        def row(r, rc):
            for j in range(JCOLS):
                sl = pl.ds(j * LANES, LANES)
                tok_v[r, sl] = tok_v[r, sl] + pos_v[r, sl] + typ_v[r, sl]
            return rc
        lax.fori_loop(0, R, row, 0)
        pltpu.sync_copy(tok_v, out_hbm.at[pl.ds(base, R)])
        return carry

    lax.fori_loop(0, NCHUNK, chunk, 0)


@jax.jit
def _run(ids, tts, tok_tab, pos_tab, typ_tab):
    mesh = plsc.VectorSubcoreMesh(core_axis_name="c", subcore_axis_name="s")
    f = pl.kernel(
        _body,
        out_type=jax.ShapeDtypeStruct((N, D), jnp.float32),
        mesh=mesh,
        scratch_types=[
            pltpu.VMEM((R,), jnp.int32),
            pltpu.VMEM((R,), jnp.int32),
            pltpu.VMEM((R, D), jnp.float32),
            pltpu.VMEM((R, D), jnp.float32),
            pltpu.VMEM((R, D), jnp.float32),
            pltpu.SemaphoreType.DMA,
            pltpu.SemaphoreType.DMA,
        ],
    )
    return f(ids, tts, tok_tab, pos_tab, typ_tab)


def kernel(input_ids, token_type_ids, token_table, pos_table, type_table):
    ids = input_ids.reshape(-1).astype(jnp.int32)
    tts = token_type_ids.reshape(-1).astype(jnp.int32)
    out = _run(ids, tts, token_table, pos_table, type_table)
    return out.reshape(B, S, D)


# speedup vs baseline: 4.5955x; 4.5955x over previous
import jax
import jax.numpy as jnp
from jax import lax
from jax.experimental import pallas as pl
from jax.experimental.pallas import tpu as pltpu
from jax.experimental.pallas import tpu_sc as plsc

B, S, D = 4, 8192, 768
N = B * S
NC, NS = 2, 16
NW = NC * NS
TPW = N // NW
R = 32
NCHUNK = TPW // R
LANES = 16
JCOLS = D // LANES
JG = 12
NG = JCOLS // JG


def _body(ids_hbm, tt_hbm, tok_tab, pos_tab, typ_tab, out_hbm,
          idx0, idx1, ttx0, ttx1, tok0, tok1, pos0, pos1, typ2,
          sem_tok0, sem_tok1, sem_pos0, sem_pos1, sem_out0, sem_out1):
    idx_v = (idx0, idx1)
    ttx_v = (ttx0, ttx1)
    tok_v = (tok0, tok1)
    pos_v = (pos0, pos1)
    sem_tok = (sem_tok0, sem_tok1)
    sem_pos = (sem_pos0, sem_pos1)
    sem_out = (sem_out0, sem_out1)

    wid = lax.axis_index("s") * NC + lax.axis_index("c")
    base0 = wid * TPW
    pos0_ = base0 % S

    pltpu.sync_copy(typ_tab, typ2)

    def start(c, bi):
        base = base0 + c * R
        pltpu.sync_copy(ids_hbm.at[pl.ds(base, R)], idx_v[bi])
        pltpu.sync_copy(tt_hbm.at[pl.ds(base, R)], ttx_v[bi])
        pltpu.async_copy(tok_tab.at[idx_v[bi]], tok_v[bi], sem_tok[bi])
        pltpu.async_copy(pos_tab.at[pl.ds(pos0_ + c * R, R)], pos_v[bi],
                         sem_pos[bi])

    def wait_in(bi):
        pltpu.make_async_copy(tok_tab.at[idx_v[bi]], tok_v[bi],
                              sem_tok[bi]).wait()
        pltpu.make_async_copy(pos_tab.at[pl.ds(0, R)], pos_v[bi],
                              sem_pos[bi]).wait()

    def fire_out(c, bi):
        base = base0 + c * R
        pltpu.async_copy(pos_v[bi], out_hbm.at[pl.ds(base, R)], sem_out[bi])

    def wait_out(bi):
        pltpu.make_async_copy(pos_v[bi], out_hbm.at[pl.ds(0, R)],
                              sem_out[bi]).wait()

    def compute(bi):
        tok, pos, ttx = tok_v[bi], pos_v[bi], ttx_v[bi]
        for g in range(NG):
            t0r = [typ2[0, pl.ds((g * JG + j) * LANES, LANES)]
                   for j in range(JG)]
            d1r = [typ2[1, pl.ds((g * JG + j) * LANES, LANES)] - t0r[j]
                   for j in range(JG)]

            def grp(rg, carry):
                tvals = ttx[pl.ds(rg * LANES, LANES)].astype(jnp.float32)
                for lane in range(LANES):
                    tf = jnp.full((LANES,), tvals[lane], jnp.float32)
                    r = rg * LANES + lane
                    for j in range(JG):
                        sl = pl.ds((g * JG + j) * LANES, LANES)
                        trow = t0r[j] + tf * d1r[j]
                        plsc.addupdate(pos.at[r, sl], tok[r, sl] + trow)
                return carry
            lax.fori_loop(0, R // LANES, grp, 0)

    start(0, 0)

    def pair(cc, carry):
        c0 = 2 * cc

        @pl.when(cc > 0)
        def _():
            wait_out(1)
        start(c0 + 1, 1)
        wait_in(0)
        compute(0)
        fire_out(c0, 0)

        @pl.when(cc < NCHUNK // 2 - 1)
        def _():
            wait_out(0)
            start(c0 + 2, 0)
        wait_in(1)
        compute(1)
        fire_out(c0 + 1, 1)
        return carry

    lax.fori_loop(0, NCHUNK // 2, pair, 0)
    wait_out(0)
    wait_out(1)


@jax.jit
def _run(ids, tts, tok_tab, pos_tab, typ_tab):
    mesh = plsc.VectorSubcoreMesh(core_axis_name="c", subcore_axis_name="s")
    f = pl.kernel(
        _body,
        out_type=jax.ShapeDtypeStruct((N, D), jnp.float32),
        mesh=mesh,
        scratch_types=[
            pltpu.VMEM((R,), jnp.int32),
            pltpu.VMEM((R,), jnp.int32),
            pltpu.VMEM((R,), jnp.int32),
            pltpu.VMEM((R,), jnp.int32),
            pltpu.VMEM((R, D), jnp.float32),
            pltpu.VMEM((R, D), jnp.float32),
            pltpu.VMEM((R, D), jnp.float32),
            pltpu.VMEM((R, D), jnp.float32),
            pltpu.VMEM((2, D), jnp.float32),
            pltpu.SemaphoreType.DMA,
            pltpu.SemaphoreType.DMA,
            pltpu.SemaphoreType.DMA,
            pltpu.SemaphoreType.DMA,
            pltpu.SemaphoreType.DMA,
            pltpu.SemaphoreType.DMA,
        ],
    )
    return f(ids, tts, tok_tab, pos_tab, typ_tab)


def kernel(input_ids, token_type_ids, token_table, pos_table, type_table):
    ids = input_ids.reshape(-1).astype(jnp.int32)
    tts = token_type_ids.reshape(-1).astype(jnp.int32)
    out = _run(ids, tts, token_table, pos_table, type_table)
    return out.reshape(B, S, D)


# speedup vs baseline: 6.0463x; 1.3157x over previous
import jax
import jax.numpy as jnp
from jax import lax
from jax.experimental import pallas as pl
from jax.experimental.pallas import tpu as pltpu
from jax.experimental.pallas import tpu_sc as plsc

B, S, D = 4, 8192, 768
N = B * S
NC, NS = 2, 16
NW = NC * NS
TPW = N // NW
R = 16
NCHUNK = TPW // R
PH = 4
LANES = 16
JCOLS = D // LANES
JG = 12
NG = JCOLS // JG


def _body(ids_hbm, tt_hbm, tok_tab, pos_tab, typ_tab, out_hbm,
          idx_all, ttx_all, typ2,
          tok0, tok1, tok2, tok3, pos0, pos1, pos2, pos3,
          st0, st1, st2, st3, sp0, sp1, sp2, sp3, so0, so1, so2, so3):
    tok = (tok0, tok1, tok2, tok3)
    pos = (pos0, pos1, pos2, pos3)
    sem_tok = (st0, st1, st2, st3)
    sem_pos = (sp0, sp1, sp2, sp3)
    sem_out = (so0, so1, so2, so3)

    wid = lax.axis_index("s") * NC + lax.axis_index("c")
    base0 = wid * TPW
    pbase0 = base0 % S

    pltpu.sync_copy(ids_hbm.at[pl.ds(base0, TPW)], idx_all)
    pltpu.sync_copy(tt_hbm.at[pl.ds(base0, TPW)], ttx_all)
    pltpu.sync_copy(typ_tab, typ2)

    def start(c, ph):
        pltpu.async_copy(tok_tab.at[idx_all.at[pl.ds(c * R, R)]],
                         tok[ph], sem_tok[ph])
        pltpu.async_copy(pos_tab.at[pl.ds(pbase0 + c * R, R)],
                         pos[ph], sem_pos[ph])

    def wait_in(ph):
        pltpu.make_async_copy(tok_tab.at[idx_all.at[pl.ds(0, R)]],
                              tok[ph], sem_tok[ph]).wait()
        pltpu.make_async_copy(pos_tab.at[pl.ds(0, R)], pos[ph],
                              sem_pos[ph]).wait()

    def fire_out(c, ph):
        pltpu.async_copy(tok[ph], out_hbm.at[pl.ds(base0 + c * R, R)],
                         sem_out[ph])

    def wait_out(ph):
        pltpu.make_async_copy(tok[ph], out_hbm.at[pl.ds(0, R)],
                              sem_out[ph]).wait()

    def compute(c, ph):
        tokb, posb = tok[ph], pos[ph]
        tvals = ttx_all[pl.ds(c * R, LANES)].astype(jnp.float32)
        for g in range(NG):
            t0r = [typ2[0, pl.ds((g * JG + j) * LANES, LANES)]
                   for j in range(JG)]
            d1r = [typ2[1, pl.ds((g * JG + j) * LANES, LANES)] - t0r[j]
                   for j in range(JG)]

            def row(r, carry):
                tf = tvals.at[jnp.full((LANES,), r, jnp.int32)].get(
                    mode="promise_in_bounds")
                for j in range(JG):
                    sl = pl.ds((g * JG + j) * LANES, LANES)
                    trow = t0r[j] + tf * d1r[j]
                    plsc.addupdate(tokb.at[r, sl], posb[r, sl] + trow)
                return carry
            lax.fori_loop(0, R, row, 0)

    start(0, 0)
    start(1, 1)

    def quad(cc, carry):
        for k in range(PH):
            c = PH * cc + k
            wait_in(k)
            compute(c, k)
            fire_out(c, k)
            k2 = (k + 2) % PH

            @pl.when(jnp.logical_and(c >= 2, c + 2 < NCHUNK))
            def _():
                wait_out(k2)

            @pl.when(c + 2 < NCHUNK)
            def _():
                start(c + 2, k2)
        return carry

    lax.fori_loop(0, NCHUNK // PH, quad, 0)
    for ph in range(PH):
        wait_out(ph)


@jax.jit
def _run(ids, tts, tok_tab, pos_tab, typ_tab):
    mesh = plsc.VectorSubcoreMesh(core_axis_name="c", subcore_axis_name="s")
    f = pl.kernel(
        _body,
        out_type=jax.ShapeDtypeStruct((N, D), jnp.float32),
        mesh=mesh,
        scratch_types=(
            [pltpu.VMEM((TPW,), jnp.int32),
             pltpu.VMEM((TPW,), jnp.int32),
             pltpu.VMEM((2, D), jnp.float32)]
            + [pltpu.VMEM((R, D), jnp.float32) for _ in range(2 * PH)]
            + [pltpu.SemaphoreType.DMA for _ in range(3 * PH)]
        ),
    )
    return f(ids, tts, tok_tab, pos_tab, typ_tab)


def kernel(input_ids, token_type_ids, token_table, pos_table, type_table):
    ids = input_ids.reshape(-1).astype(jnp.int32)
    tts = token_type_ids.reshape(-1).astype(jnp.int32)
    out = _run(ids, tts, token_table, pos_table, type_table)
    return out.reshape(B, S, D)
